# baseline (speedup 1.0000x reference)
import jax
import jax.numpy as jnp
from jax import lax
from jax.experimental import pallas as pl
from jax.experimental.pallas import tpu as pltpu
from jax.experimental.pallas import tpu_sc as plsc

N = 100000
C0 = 256
C1 = 128
M1 = 5
F1 = M1 * C1
LANES = 16
B = 80
NB = N // B


def _sc_body(f0_hbm, f1_hbm, types_hbm, emb0_hbm, emb1_hbm,
             out0_hbm, out1_hbm,
             idx_v, e0_v, e1_v, f0_v, f1_v, sem):
    info = plsc.get_sparse_core_info()
    nc = info.num_cores
    nw = nc * info.num_subcores
    wid = lax.axis_index("s") * nc + lax.axis_index("c")
    nblk = (NB - wid + nw - 1) // nw

    def block_body(i, carry):
        base = (wid + i * nw) * B
        pltpu.sync_copy(types_hbm.at[pl.ds(base, B)], idx_v)
        pltpu.async_copy(emb0_hbm.at[idx_v], e0_v, sem).wait()
        pltpu.async_copy(emb1_hbm.at[idx_v], e1_v, sem).wait()
        pltpu.sync_copy(f0_hbm.at[pl.ds(base, B)], f0_v)
        pltpu.sync_copy(f1_hbm.at[pl.ds(base, B)], f1_v)

        def row_body(r, rc):
            for c in range(C0 // LANES):
                sl = pl.ds(c * LANES, LANES)
                f0_v[r, sl] = f0_v[r, sl] * e0_v[r, sl]
            for c in range(C1 // LANES):
                erow = e1_v[r, pl.ds(c * LANES, LANES)]
                for m in range(M1):
                    sl = pl.ds(m * C1 + c * LANES, LANES)
                    f1_v[r, sl] = f1_v[r, sl] * erow
            return rc

        lax.fori_loop(0, B, row_body, 0)
        pltpu.sync_copy(f0_v, out0_hbm.at[pl.ds(base, B)])
        pltpu.sync_copy(f1_v, out1_hbm.at[pl.ds(base, B)])
        return carry

    lax.fori_loop(0, nblk, block_body, 0)


def kernel(features_0, features_1, atomic_types, emb_0, emb_1):
    f1_flat = features_1.reshape(N, F1)
    types32 = atomic_types.astype(jnp.int32)
    k = pl.kernel(
        _sc_body,
        mesh=plsc.VectorSubcoreMesh(core_axis_name="c", subcore_axis_name="s"),
        out_type=[
            jax.ShapeDtypeStruct((N, C0), jnp.float32),
            jax.ShapeDtypeStruct((N, F1), jnp.float32),
        ],
        scratch_types=[
            pltpu.VMEM((B,), jnp.int32),
            pltpu.VMEM((B, C0), jnp.float32),
            pltpu.VMEM((B, C1), jnp.float32),
            pltpu.VMEM((B, C0), jnp.float32),
            pltpu.VMEM((B, F1), jnp.float32),
            pltpu.SemaphoreType.DMA,
        ],
    )
    out0, out1 = k(features_0, f1_flat, types32, emb_0, emb_1)
    return (out0, out1.reshape(N, M1, C1))


# speedup vs baseline: 1.5894x; 1.5894x over previous
import jax
import jax.numpy as jnp
from jax import lax
from jax.experimental import pallas as pl
from jax.experimental.pallas import tpu as pltpu
from jax.experimental.pallas import tpu_sc as plsc

N = 100000
C0 = 256
C1 = 128
M1 = 5
LANES = 16
B = 16
NB = N // B
NW = 32
BLK_PER, BLK_REM = NB // NW, NB % NW
IDX_MAIN = BLK_PER * B
IDX_MAX = (BLK_PER + 1) * B


def _sc_body(f0_hbm, f1_hbm, types_hbm, emb0_hbm, emb1_hbm,
             out0_hbm, out1_hbm,
             idx_all,
             e0a, e0b, e0c, e1a, e1b, e1c,
             f0a, f0b, f0c, f1a, f1b, f1c,
             sia, sib, sic, soa, sob, soc):
    info = plsc.get_sparse_core_info()
    nc = info.num_cores
    wid = lax.axis_index("s") * nc + lax.axis_index("c")

    e0 = (e0a, e0b, e0c)
    e1 = (e1a, e1b, e1c)
    f0 = (f0a, f0b, f0c)
    f1 = (f1a, f1b, f1c)
    sin = (sia, sib, sic)
    sout = (soa, sob, soc)

    count = BLK_PER + jnp.where(wid < BLK_REM, 1, 0)
    start_blk = BLK_PER * wid + jnp.minimum(wid, BLK_REM)
    start_atom = start_blk * B

    pltpu.sync_copy(types_hbm.at[pl.ds(start_atom, IDX_MAIN)],
                    idx_all.at[pl.ds(0, IDX_MAIN)])

    @pl.when(count == BLK_PER + 1)
    def _():
        pltpu.sync_copy(types_hbm.at[pl.ds(start_atom + IDX_MAIN, B)],
                        idx_all.at[pl.ds(IDX_MAIN, B)])

    def in_descs(j, r):
        base = (start_blk + j) * B
        idx_ref = idx_all.at[pl.ds(j * B, B)]
        return (
            pltpu.make_async_copy(emb0_hbm.at[idx_ref], e0[r], sin[r]),
            pltpu.make_async_copy(emb1_hbm.at[idx_ref], e1[r], sin[r]),
            pltpu.make_async_copy(f0_hbm.at[pl.ds(base, B)], f0[r], sin[r]),
            pltpu.make_async_copy(f1_hbm.at[pl.ds(base, B)], f1[r], sin[r]),
        )

    def out_descs(j, r):
        base = (start_blk + j) * B
        return (
            pltpu.make_async_copy(f0[r], out0_hbm.at[pl.ds(base, B)], sout[r]),
            pltpu.make_async_copy(f1[r], out1_hbm.at[pl.ds(base, B)], sout[r]),
        )

    def prefetch(j, r):
        for d in in_descs(j, r):
            d.start()

    def wait_in(j, r):
        for d in in_descs(j, r):
            d.wait()

    def issue_out(j, r):
        for d in out_descs(j, r):
            d.start()

    def wait_out(j, r):
        for d in out_descs(j, r):
            d.wait()

    def compute(r):
        f0r, e0r, f1r, e1r = f0[r], e0[r], f1[r], e1[r]

        def row_body(row, rc):
            for c in range(C0 // LANES):
                sl = pl.ds(c * LANES, LANES)
                f0r[row, sl] = f0r[row, sl] * e0r[row, sl]
            for c in range(C1 // LANES):
                sl = pl.ds(c * LANES, LANES)
                erow = e1r[row, sl]
                for m in range(M1):
                    f1r[row, m, sl] = f1r[row, m, sl] * erow
            return rc

        lax.fori_loop(0, B, row_body, 0)

    prefetch(0, 0)

    def outer(g, carry):
        for b in range(3):
            j = g * 3 + b
            rn = (b + 1) % 3

            @pl.when(j + 1 < count)
            def _():
                @pl.when(j >= 2)
                def _():
                    wait_out(j - 2, rn)
                prefetch(j + 1, rn)

            @pl.when(j < count)
            def _():
                wait_in(j, b)
                compute(b)
                issue_out(j, b)
        return carry

    lax.fori_loop(0, (count + 2) // 3, outer, 0)

    for r in range(3):
        last = jnp.where((count - 1) % 3 == r, count - 1, count - 2)
        pending = jnp.logical_or((count - 1) % 3 == r, (count - 2) % 3 == r)

        @pl.when(pending)
        def _():
            wait_out(last, r)


def kernel(features_0, features_1, atomic_types, emb_0, emb_1):
    types32 = atomic_types.astype(jnp.int32)
    k = pl.kernel(
        _sc_body,
        mesh=plsc.VectorSubcoreMesh(core_axis_name="c", subcore_axis_name="s"),
        out_type=[
            jax.ShapeDtypeStruct((N, C0), jnp.float32),
            jax.ShapeDtypeStruct((N, M1, C1), jnp.float32),
        ],
        scratch_types=(
            [pltpu.VMEM((IDX_MAX,), jnp.int32)]
            + [pltpu.VMEM((B, C0), jnp.float32) for _ in range(3)]
            + [pltpu.VMEM((B, C1), jnp.float32) for _ in range(3)]
            + [pltpu.VMEM((B, C0), jnp.float32) for _ in range(3)]
            + [pltpu.VMEM((B, M1, C1), jnp.float32) for _ in range(3)]
            + [pltpu.SemaphoreType.DMA for _ in range(6)]
        ),
    )
    out0, out1 = k(features_0, features_1, types32, emb_0, emb_1)
    return (out0, out1)
